# baseline (speedup 1.0000x reference)
import functools

import jax
import jax.numpy as jnp
from jax import lax
from jax.experimental import pallas as pl
from jax.experimental.pallas import tpu as pltpu
from jax.experimental.pallas import tpu_sc as plsc

_N = 50000
_E = 1600000
_CUTOFF = 10.0
_KE = 14.399645351950548

_NC = 2
_NS = 16
_NW = _NC * _NS
_EPW = _E // _NW
_CH = 2000
_NCHUNK = _EPW // _CH
_NVEC = _CH // 16

_WBLK = 160000


def _chi_tc(d):
    x = d * (2.0 / _CUTOFF)
    x3 = x * x * x
    x4 = x3 * x
    x5 = x4 * x
    poly = 1.0 - 6.0 * x5 + 15.0 * x4 - 10.0 * x3
    p = jnp.where(d < (_CUTOFF * 0.5), poly, 0.0)
    return p / jnp.sqrt(d * d + 1.0) + (1.0 - p) / d


def _pre_body(idx_ref, d_ref, src_ref, dst_ref, w_ref):
    src_ref[...] = idx_ref[0, :]
    dst_ref[...] = idx_ref[1, :]
    w_ref[...] = _chi_tc(d_ref[...])


def _sc_body(qi_hbm, src_hbm, dst_hbm, w_hbm, out_hbm,
             qi_v, acc_v,
             src_a, dst_a, w_a, src_b, dst_b, w_b,
             semq, sema, semb):
    wid = lax.axis_index("c") * _NS + lax.axis_index("s")
    base = wid * _EPW
    buf_a = (src_a, dst_a, w_a)
    buf_b = (src_b, dst_b, w_b)

    def issue(c, bufs, sem):
        off = base + c * _CH
        pltpu.async_copy(src_hbm.at[pl.ds(off, _CH)], bufs[0], sem)
        pltpu.async_copy(dst_hbm.at[pl.ds(off, _CH)], bufs[1], sem)
        pltpu.async_copy(w_hbm.at[pl.ds(off, _CH)], bufs[2], sem)

    def drain(c, bufs, sem):
        off = base + c * _CH
        pltpu.make_async_copy(src_hbm.at[pl.ds(off, _CH)], bufs[0], sem).wait()
        pltpu.make_async_copy(dst_hbm.at[pl.ds(off, _CH)], bufs[1], sem).wait()
        pltpu.make_async_copy(w_hbm.at[pl.ds(off, _CH)], bufs[2], sem).wait()

    def compute(bufs):
        @plsc.parallel_loop(0, _NVEC, unroll=25)
        def _vec(j):
            s = bufs[0][pl.ds(j * 16, 16)]
            t = bufs[1][pl.ds(j * 16, 16)]
            w = bufs[2][pl.ds(j * 16, 16)]
            q1 = plsc.load_gather(qi_v, [s])
            q2 = plsc.load_gather(qi_v, [t])
            plsc.addupdate_scatter(acc_v, [s], q1 * q2 * w)

    qi_cp = pltpu.async_copy(qi_hbm, qi_v, semq)
    issue(0, buf_a, sema)

    @plsc.parallel_loop(0, _N // 16, unroll=5)
    def _zero(j):
        acc_v[pl.ds(j * 16, 16)] = jnp.zeros((16,), jnp.float32)

    qi_cp.wait()

    def chunk_pair(k, carry):
        c = 2 * k
        drain(c, buf_a, sema)
        issue(c + 1, buf_b, semb)
        compute(buf_a)
        drain(c + 1, buf_b, semb)
        issue(c + 2, buf_a, sema)
        compute(buf_b)
        return carry

    lax.fori_loop(0, (_NCHUNK - 1) // 2, chunk_pair, 0)
    drain(_NCHUNK - 1, buf_a, sema)
    compute(buf_a)
    pltpu.sync_copy(acc_v, out_hbm.at[wid])


@functools.lru_cache(maxsize=1)
def _sc_call():
    return functools.partial(
        pl.kernel,
        mesh=plsc.VectorSubcoreMesh(core_axis_name="c", subcore_axis_name="s"),
        out_type=jax.ShapeDtypeStruct((_NW, _N), jnp.float32),
        compiler_params=pltpu.CompilerParams(needs_layout_passes=False),
        scratch_types=[
            pltpu.VMEM((_N,), jnp.float32),
            pltpu.VMEM((_N,), jnp.float32),
            pltpu.VMEM((_CH,), jnp.int32),
            pltpu.VMEM((_CH,), jnp.int32),
            pltpu.VMEM((_CH,), jnp.float32),
            pltpu.VMEM((_CH,), jnp.int32),
            pltpu.VMEM((_CH,), jnp.int32),
            pltpu.VMEM((_CH,), jnp.float32),
            pltpu.SemaphoreType.DMA,
            pltpu.SemaphoreType.DMA,
            pltpu.SemaphoreType.DMA,
        ],
    )(_sc_body)


def _tc_reduce(x_ref, o_ref):
    o_ref[...] = jnp.sum(x_ref[...], axis=0) * (_KE * 0.5)


@jax.jit
def kernel(qi, edge_dist, edge_index):
    src, dst, w = pl.pallas_call(
        _pre_body,
        out_shape=(
            jax.ShapeDtypeStruct((_E,), jnp.int32),
            jax.ShapeDtypeStruct((_E,), jnp.int32),
            jax.ShapeDtypeStruct((_E,), jnp.float32),
        ),
    )(edge_index, edge_dist)
    part = _sc_call()(qi, src, dst, w)
    return pl.pallas_call(
        _tc_reduce,
        out_shape=jax.ShapeDtypeStruct((_N,), jnp.float32),
    )(part)


# speedup vs baseline: 1.1184x; 1.1184x over previous
import functools

import jax
import jax.numpy as jnp
from jax import lax
from jax.experimental import pallas as pl
from jax.experimental.pallas import tpu as pltpu
from jax.experimental.pallas import tpu_sc as plsc

_N = 50000
_E = 1600000
_CUTOFF = 10.0
_KE = 14.399645351950548

_NC = 2
_NS = 16
_NW = _NC * _NS
_CH = 2560
_NCHUNK = _E // _CH
_KMIN = _NCHUNK // _NW
_NW_EXTRA = _NCHUNK - _KMIN * _NW
_NVEC = _CH // 16


def _chi_tc(d):
    x = d * (2.0 / _CUTOFF)
    x3 = x * x * x
    x4 = x3 * x
    x5 = x4 * x
    poly = 1.0 - 6.0 * x5 + 15.0 * x4 - 10.0 * x3
    p = jnp.where(d < (_CUTOFF * 0.5), poly, 0.0)
    return p / jnp.sqrt(d * d + 1.0) + (1.0 - p) / d


def _chi_body(d_ref, w_ref):
    w_ref[...] = _chi_tc(d_ref[...])


def _sc_body(qi_hbm, edge_hbm, w_hbm, out_hbm,
             qi_v, acc_v,
             edge_a, w_a, edge_b, w_b,
             semq, sema, semb):
    wid = lax.axis_index("c") * _NS + lax.axis_index("s")
    buf_a = (edge_a, w_a)
    buf_b = (edge_b, w_b)

    def issue(k, bufs, sem):
        off = (wid + k * _NW) * _CH
        pltpu.async_copy(edge_hbm.at[:, pl.ds(off, _CH)], bufs[0], sem)
        pltpu.async_copy(w_hbm.at[pl.ds(off, _CH)], bufs[1], sem)

    def drain(k, bufs, sem):
        off = (wid + k * _NW) * _CH
        pltpu.make_async_copy(edge_hbm.at[:, pl.ds(off, _CH)], bufs[0], sem).wait()
        pltpu.make_async_copy(w_hbm.at[pl.ds(off, _CH)], bufs[1], sem).wait()

    def compute(bufs):
        @plsc.parallel_loop(0, _NVEC, unroll=5)
        def _vec(j):
            s = bufs[0][0, pl.ds(j * 16, 16)]
            t = bufs[0][1, pl.ds(j * 16, 16)]
            w = bufs[1][pl.ds(j * 16, 16)]
            q1 = plsc.load_gather(qi_v, [s])
            q2 = plsc.load_gather(qi_v, [t])
            plsc.addupdate_scatter(acc_v, [s], q1 * q2 * w)

    qi_cp = pltpu.async_copy(qi_hbm, qi_v, semq)
    issue(0, buf_a, sema)

    @plsc.parallel_loop(0, _N // 16, unroll=5)
    def _zero(j):
        acc_v[pl.ds(j * 16, 16)] = jnp.zeros((16,), jnp.float32)

    qi_cp.wait()

    def chunk_pair(p, carry):
        k = 2 * p
        drain(k, buf_a, sema)
        issue(k + 1, buf_b, semb)
        compute(buf_a)
        drain(k + 1, buf_b, semb)
        issue(k + 2, buf_a, sema)
        compute(buf_b)
        return carry

    lax.fori_loop(0, (_KMIN - 1) // 2, chunk_pair, 0)

    drain(_KMIN - 1, buf_a, sema)

    @pl.when(wid < _NW_EXTRA)
    def _issue_tail():
        issue(_KMIN, buf_b, semb)

    compute(buf_a)

    @pl.when(wid < _NW_EXTRA)
    def _tail():
        drain(_KMIN, buf_b, semb)
        compute(buf_b)

    pltpu.sync_copy(acc_v, out_hbm.at[wid])


@functools.lru_cache(maxsize=1)
def _sc_call():
    return functools.partial(
        pl.kernel,
        mesh=plsc.VectorSubcoreMesh(core_axis_name="c", subcore_axis_name="s"),
        out_type=jax.ShapeDtypeStruct((_NW, _N), jnp.float32),
        compiler_params=pltpu.CompilerParams(needs_layout_passes=False),
        scratch_types=[
            pltpu.VMEM((_N,), jnp.float32),
            pltpu.VMEM((_N,), jnp.float32),
            pltpu.VMEM((2, _CH), jnp.int32),
            pltpu.VMEM((_CH,), jnp.float32),
            pltpu.VMEM((2, _CH), jnp.int32),
            pltpu.VMEM((_CH,), jnp.float32),
            pltpu.SemaphoreType.DMA,
            pltpu.SemaphoreType.DMA,
            pltpu.SemaphoreType.DMA,
        ],
    )(_sc_body)


def _tc_reduce(x_ref, o_ref):
    o_ref[...] = jnp.sum(x_ref[...], axis=0) * (_KE * 0.5)


@jax.jit
def kernel(qi, edge_dist, edge_index):
    w = pl.pallas_call(
        _chi_body,
        out_shape=jax.ShapeDtypeStruct((_E,), jnp.float32),
    )(edge_dist)
    part = _sc_call()(qi, edge_index, w)
    return pl.pallas_call(
        _tc_reduce,
        out_shape=jax.ShapeDtypeStruct((_N,), jnp.float32),
    )(part)


# speedup vs baseline: 1.1451x; 1.0238x over previous
import functools

import jax
import jax.numpy as jnp
from jax import lax
from jax.experimental import pallas as pl
from jax.experimental.pallas import tpu as pltpu
from jax.experimental.pallas import tpu_sc as plsc

_N = 50000
_E = 1600000
_CUTOFF = 10.0
_KE = 14.399645351950548

_NC = 2
_NS = 16
_NW = _NC * _NS
_CH = 2560
_NCHUNK = _E // _CH
_KMIN = _NCHUNK // _NW
_NW_EXTRA = _NCHUNK - _KMIN * _NW
_NVEC = _CH // 16


def _chi_tc(d):
    x = d * (2.0 / _CUTOFF)
    x3 = x * x * x
    x4 = x3 * x
    x5 = x4 * x
    poly = 1.0 - 6.0 * x5 + 15.0 * x4 - 10.0 * x3
    p = jnp.where(d < (_CUTOFF * 0.5), poly, 0.0)
    dd = d * d
    return p * lax.rsqrt(dd + 1.0) + (1.0 - p) * lax.rsqrt(dd)


_CBLK = 160000
_CNCH = _E // _CBLK


def _chi_body(d_hbm, w_hbm, d0, d1, w0, w1, si0, si1, so0, so1):
    d_bufs, w_bufs = (d0, d1), (w0, w1)
    si, so = (si0, si1), (so0, so1)

    def copy_in(c, b):
        return pltpu.make_async_copy(d_hbm.at[pl.ds(c * _CBLK, _CBLK)], d_bufs[b], si[b])

    def copy_out(c, b):
        return pltpu.make_async_copy(w_bufs[b], w_hbm.at[pl.ds(c * _CBLK, _CBLK)], so[b])

    copy_in(0, 0).start()
    for c in range(_CNCH):
        b = c % 2
        if c + 1 < _CNCH:
            copy_in(c + 1, 1 - b).start()
        copy_in(c, b).wait()
        if c >= 2:
            copy_out(c - 2, b).wait()
        w_bufs[b][...] = _chi_tc(d_bufs[b][...])
        copy_out(c, b).start()
    copy_out(_CNCH - 2, _CNCH % 2).wait()
    copy_out(_CNCH - 1, (_CNCH - 1) % 2).wait()


def _sc_body(qi_hbm, edge_hbm, w_hbm, out_hbm,
             qi_v, acc_v,
             edge_a, w_a, edge_b, w_b,
             semq, sema, semb):
    wid = lax.axis_index("c") * _NS + lax.axis_index("s")
    buf_a = (edge_a, w_a)
    buf_b = (edge_b, w_b)

    def issue(k, bufs, sem):
        off = (wid + k * _NW) * _CH
        pltpu.async_copy(edge_hbm.at[:, pl.ds(off, _CH)], bufs[0], sem)
        pltpu.async_copy(w_hbm.at[pl.ds(off, _CH)], bufs[1], sem)

    def drain(k, bufs, sem):
        off = (wid + k * _NW) * _CH
        pltpu.make_async_copy(edge_hbm.at[:, pl.ds(off, _CH)], bufs[0], sem).wait()
        pltpu.make_async_copy(w_hbm.at[pl.ds(off, _CH)], bufs[1], sem).wait()

    def compute(bufs):
        @plsc.parallel_loop(0, _NVEC, unroll=5)
        def _vec(j):
            s = bufs[0][0, pl.ds(j * 16, 16)]
            t = bufs[0][1, pl.ds(j * 16, 16)]
            w = bufs[1][pl.ds(j * 16, 16)]
            q1 = plsc.load_gather(qi_v, [s])
            q2 = plsc.load_gather(qi_v, [t])
            plsc.addupdate_scatter(acc_v, [s], q1 * q2 * w)

    qi_cp = pltpu.async_copy(qi_hbm, qi_v, semq)
    issue(0, buf_a, sema)

    @plsc.parallel_loop(0, _N // 16, unroll=5)
    def _zero(j):
        acc_v[pl.ds(j * 16, 16)] = jnp.zeros((16,), jnp.float32)

    qi_cp.wait()

    def chunk_pair(p, carry):
        k = 2 * p
        drain(k, buf_a, sema)
        issue(k + 1, buf_b, semb)
        compute(buf_a)
        drain(k + 1, buf_b, semb)
        issue(k + 2, buf_a, sema)
        compute(buf_b)
        return carry

    lax.fori_loop(0, (_KMIN - 1) // 2, chunk_pair, 0)

    drain(_KMIN - 1, buf_a, sema)

    @pl.when(wid < _NW_EXTRA)
    def _issue_tail():
        issue(_KMIN, buf_b, semb)

    compute(buf_a)

    @pl.when(wid < _NW_EXTRA)
    def _tail():
        drain(_KMIN, buf_b, semb)
        compute(buf_b)

    pltpu.sync_copy(acc_v, out_hbm.at[wid])


@functools.lru_cache(maxsize=1)
def _sc_call():
    return functools.partial(
        pl.kernel,
        mesh=plsc.VectorSubcoreMesh(core_axis_name="c", subcore_axis_name="s"),
        out_type=jax.ShapeDtypeStruct((_NW, _N), jnp.float32),
        compiler_params=pltpu.CompilerParams(needs_layout_passes=False),
        scratch_types=[
            pltpu.VMEM((_N,), jnp.float32),
            pltpu.VMEM((_N,), jnp.float32),
            pltpu.VMEM((2, _CH), jnp.int32),
            pltpu.VMEM((_CH,), jnp.float32),
            pltpu.VMEM((2, _CH), jnp.int32),
            pltpu.VMEM((_CH,), jnp.float32),
            pltpu.SemaphoreType.DMA,
            pltpu.SemaphoreType.DMA,
            pltpu.SemaphoreType.DMA,
        ],
    )(_sc_body)


def _tc_reduce(x_ref, o_ref):
    o_ref[...] = jnp.sum(x_ref[...], axis=0) * (_KE * 0.5)


@jax.jit
def kernel(qi, edge_dist, edge_index):
    w = pl.pallas_call(
        _chi_body,
        in_specs=[pl.BlockSpec(memory_space=pl.ANY)],
        out_specs=pl.BlockSpec(memory_space=pl.ANY),
        out_shape=jax.ShapeDtypeStruct((_E,), jnp.float32),
        scratch_shapes=[
            pltpu.VMEM((_CBLK,), jnp.float32),
            pltpu.VMEM((_CBLK,), jnp.float32),
            pltpu.VMEM((_CBLK,), jnp.float32),
            pltpu.VMEM((_CBLK,), jnp.float32),
            pltpu.SemaphoreType.DMA,
            pltpu.SemaphoreType.DMA,
            pltpu.SemaphoreType.DMA,
            pltpu.SemaphoreType.DMA,
        ],
    )(edge_dist)
    part = _sc_call()(qi, edge_index, w)
    return pl.pallas_call(
        _tc_reduce,
        out_shape=jax.ShapeDtypeStruct((_N,), jnp.float32),
    )(part)
